# Initial kernel scaffold; baseline (speedup 1.0000x reference)
import jax
import jax.numpy as jnp
from jax.experimental import pallas as pl


def kernel(h, node_feat_discrete, segment_ids, W, b, scale_table, bias_table, mean, std):
    raise NotImplementedError("write your pallas kernel here")


# speedup vs baseline: 9.4637x; 9.4637x over previous
import functools

import jax
import jax.numpy as jnp
from jax import lax
from jax.experimental import pallas as pl
from jax.experimental.pallas import tpu as pltpu
from jax.experimental.pallas import tpu_sc as plsc

N = 100000
D = 128
S = 256
U = 100
AW = 144
ACC = S * AW
C = 160
NCH = N // C
GPC = C // 16
NW = 32

_mesh = plsc.VectorSubcoreMesh(core_axis_name="c", subcore_axis_name="s")
_params = pltpu.CompilerParams(needs_layout_passes=False)


@functools.partial(
    pl.kernel,
    out_type=jax.ShapeDtypeStruct((NW, ACC), jnp.float32),
    mesh=_mesh,
    compiler_params=_params,
    scratch_types=[
        pltpu.VMEM((C * D,), jnp.float32),
        pltpu.VMEM((C,), jnp.int32),
        pltpu.VMEM((C,), jnp.int32),
        pltpu.VMEM((D,), jnp.float32),
        pltpu.VMEM((D,), jnp.float32),
        pltpu.VMEM((D,), jnp.float32),
        pltpu.VMEM((ACC,), jnp.float32),
        pltpu.VMEM((S,), jnp.float32),
    ],
)
def _phase1(h_hbm, seg_hbm, typ_hbm, w_hbm, a_hbm, c_hbm, part_hbm,
            hbuf, segbuf, typbuf, wbuf, abuf, cbuf, acc, dsc):
    wid = lax.axis_index("s") * 2 + lax.axis_index("c")
    pltpu.sync_copy(w_hbm, wbuf)
    pltpu.sync_copy(a_hbm, abuf)
    pltpu.sync_copy(c_hbm, cbuf)

    zeros16 = jnp.zeros((16,), jnp.float32)

    def zero_body(i, carry):
        acc[pl.ds(i * 16, 16)] = zeros16
        return carry

    lax.fori_loop(0, ACC // 16, zero_body, 0)

    wvecs = [wbuf[pl.ds(16 * j, 16)] for j in range(8)]
    iota16 = lax.iota(jnp.int32, 16)

    nchunks = (NCH - wid + NW - 1) // NW

    def chunk_body(k, carry):
        chunk = wid + k * NW
        off = chunk * C
        pltpu.sync_copy(seg_hbm.at[pl.ds(off, C)], segbuf)
        pltpu.sync_copy(typ_hbm.at[pl.ds(off, C)], typbuf)
        pltpu.sync_copy(h_hbm.at[pl.ds(chunk * (C * D), C * D)], hbuf)

        def group_body(g, gcarry):
            gb = g * (16 * D)
            seg_vec = segbuf[pl.ds(g * 16, 16)]
            typ_vec = typbuf[pl.ds(g * 16, 16)]
            a_g = plsc.load_gather(abuf, [typ_vec])
            c_g = plsc.load_gather(cbuf, [typ_vec])
            for r in range(16):
                rb = gb + r * D
                hv = [hbuf[pl.ds(rb + 16 * j, 16)] for j in range(8)]
                m = hv[0] * wvecs[0]
                for j in range(1, 8):
                    m = m + hv[j] * wvecs[j]
                plsc.store_scatter(dsc, [iota16 * 16 + r], m)
                seg_r = seg_vec[r]
                ab = seg_r * AW
                for j in range(8):
                    plsc.addupdate(acc.at[pl.ds(ab + 16 * j, 16)], hv[j])
            dv = dsc[pl.ds(0, 16)]
            for l in range(1, 16):
                dv = dv + dsc[pl.ds(16 * l, 16)]
            f_vec = a_g * dv + c_g
            plsc.addupdate_scatter(acc, [seg_vec * AW + D], f_vec)
            return gcarry

        lax.fori_loop(0, GPC, group_body, 0)
        return carry

    lax.fori_loop(0, nchunks, chunk_body, 0)
    pltpu.sync_copy(acc, part_hbm.at[wid])


@functools.partial(
    pl.kernel,
    out_type=jax.ShapeDtypeStruct((ACC,), jnp.float32),
    mesh=_mesh,
    compiler_params=_params,
    scratch_types=[
        pltpu.VMEM((NW * 8 * AW,), jnp.float32),
        pltpu.VMEM((8 * AW,), jnp.float32),
        pltpu.SemaphoreType.DMA,
    ],
)
def _phase2(part_hbm, out_hbm, pbuf, obuf, sem):
    wid = lax.axis_index("s") * 2 + lax.axis_index("c")
    base = wid * (8 * AW)
    descs = []
    for p in range(NW):
        descs.append(pltpu.async_copy(
            part_hbm.at[p, pl.ds(base, 8 * AW)],
            pbuf.at[pl.ds(p * 8 * AW, 8 * AW)], sem))
    for dsx in descs:
        dsx.wait()
    zeros16 = jnp.zeros((16,), jnp.float32)
    for i in range(8 * AW // 16):
        obuf[pl.ds(16 * i, 16)] = zeros16

    def p_body(p, carry):
        pb = p * (8 * AW)
        for i in range(8 * AW // 16):
            plsc.addupdate(obuf.at[pl.ds(16 * i, 16)],
                           pbuf[pl.ds(pb + 16 * i, 16)])
        return carry

    lax.fori_loop(0, NW, p_body, 0)
    pltpu.sync_copy(obuf, out_hbm.at[pl.ds(base, 8 * AW)])


def kernel(h, node_feat_discrete, segment_ids, W, b, scale_table, bias_table,
           mean, std):
    h = h.astype(jnp.float32)
    seg = segment_ids.astype(jnp.int32)
    typ = node_feat_discrete.astype(jnp.int32)
    std0 = std.astype(jnp.float32)[0]
    a_small = std0 * scale_table.astype(jnp.float32)[:, 0]
    c_small = (a_small * b.astype(jnp.float32)[0]
               + std0 * bias_table.astype(jnp.float32)[:, 0]
               + mean.astype(jnp.float32)[0])
    a_pad = jnp.zeros((D,), jnp.float32).at[:U].set(a_small)
    c_pad = jnp.zeros((D,), jnp.float32).at[:U].set(c_small)
    partials = _phase1(h.reshape(-1), seg, typ,
                       W.astype(jnp.float32).reshape(-1), a_pad, c_pad)
    outflat = _phase2(partials)
    r = outflat.reshape(S, AW)
    return (r[:, :D], r[:, D:D + 1])


# speedup vs baseline: 15.5663x; 1.6448x over previous
import functools

import jax
import jax.numpy as jnp
from jax import lax
from jax.experimental import pallas as pl
from jax.experimental.pallas import tpu as pltpu
from jax.experimental.pallas import tpu_sc as plsc

N = 100000
D = 128
S = 256
U = 100
AW = 144
ACC = S * AW
C = 160
NCH = N // C
GPC = C // 16
NW = 32

_mesh = plsc.VectorSubcoreMesh(core_axis_name="c", subcore_axis_name="s")
_params = pltpu.CompilerParams(needs_layout_passes=False)


NBUF = 2
MAXCH = (NCH + NW - 1) // NW


@functools.partial(
    pl.kernel,
    out_type=jax.ShapeDtypeStruct((NW, ACC), jnp.float32),
    mesh=_mesh,
    compiler_params=_params,
    scratch_types=[
        pltpu.VMEM((NBUF * C * D,), jnp.float32),
        pltpu.VMEM((NBUF * C,), jnp.int32),
        pltpu.VMEM((NBUF * C,), jnp.int32),
        pltpu.VMEM((D,), jnp.float32),
        pltpu.VMEM((D,), jnp.float32),
        pltpu.VMEM((D,), jnp.float32),
        pltpu.VMEM((ACC,), jnp.float32),
        pltpu.VMEM((GPC * S,), jnp.float32),
        pltpu.SemaphoreType.DMA((NBUF,)),
        pltpu.SemaphoreType.DMA((NBUF,)),
        pltpu.SemaphoreType.DMA((NBUF,)),
    ],
)
def _phase1(h_hbm, seg_hbm, typ_hbm, w_hbm, a_hbm, c_hbm, part_hbm,
            hbuf, segbuf, typbuf, wbuf, abuf, cbuf, acc, dsc,
            hsem, ssem, tsem):
    wid = lax.axis_index("s") * 2 + lax.axis_index("c")
    pltpu.sync_copy(w_hbm, wbuf)
    pltpu.sync_copy(a_hbm, abuf)
    pltpu.sync_copy(c_hbm, cbuf)

    zeros16 = jnp.zeros((16,), jnp.float32)

    def zero_body(i, carry):
        acc[pl.ds(i * 16, 16)] = zeros16
        return carry

    lax.fori_loop(0, ACC // 16, zero_body, 0)

    wvecs = [wbuf[pl.ds(16 * j, 16)] for j in range(8)]
    iota16 = lax.iota(jnp.int32, 16)
    dsc_idx = iota16 * 16

    nchunks = (NCH - wid + NW - 1) // NW

    def _issue(b, i):
        chunk = wid + i * NW
        off = chunk * C
        pltpu.async_copy(seg_hbm.at[pl.ds(off, C)],
                         segbuf.at[pl.ds(b * C, C)], ssem.at[b])
        pltpu.async_copy(typ_hbm.at[pl.ds(off, C)],
                         typbuf.at[pl.ds(b * C, C)], tsem.at[b])
        pltpu.async_copy(h_hbm.at[pl.ds(chunk * (C * D), C * D)],
                         hbuf.at[pl.ds(b * C * D, C * D)], hsem.at[b])

    def _wait(b):
        pltpu.make_async_copy(seg_hbm.at[pl.ds(0, C)],
                              segbuf.at[pl.ds(b * C, C)], ssem.at[b]).wait()
        pltpu.make_async_copy(typ_hbm.at[pl.ds(0, C)],
                              typbuf.at[pl.ds(b * C, C)], tsem.at[b]).wait()
        pltpu.make_async_copy(h_hbm.at[pl.ds(0, C * D)],
                              hbuf.at[pl.ds(b * C * D, C * D)], hsem.at[b]).wait()

    def _process(b):
        hb0 = b * C * D
        sb0 = b * C

        @plsc.parallel_loop(0, C, 1, unroll=2)
        def row_body(r):
            rb = hb0 + r * D
            seg_b = plsc.load_gather(
                segbuf, [jnp.full((16,), sb0 + r, jnp.int32)])
            b0 = seg_b * AW + iota16
            prods = []
            for j in range(8):
                hv = hbuf[pl.ds(rb + 16 * j, 16)]
                plsc.addupdate_scatter(acc, [b0 + (16 * j)], hv)
                prods.append(hv * wvecs[j])
            m = ((prods[0] + prods[1]) + (prods[2] + prods[3])) + \
                ((prods[4] + prods[5]) + (prods[6] + prods[7]))
            base_s = ((r >> 4) << 8) | (r & 15)
            plsc.store_scatter(dsc, [dsc_idx + base_s], m)

        @plsc.parallel_loop(0, GPC, 1)
        def group_body(g):
            seg_vec = segbuf[pl.ds(sb0 + g * 16, 16)]
            typ_vec = typbuf[pl.ds(sb0 + g * 16, 16)]
            a_g = plsc.load_gather(abuf, [typ_vec])
            c_g = plsc.load_gather(cbuf, [typ_vec])
            db = g * S
            vs = [dsc[pl.ds(db + 16 * l, 16)] for l in range(16)]
            while len(vs) > 1:
                vs = [vs[i] + vs[i + 1] for i in range(0, len(vs), 2)]
            f_vec = a_g * vs[0] + c_g
            plsc.addupdate_scatter(acc, [seg_vec * AW + D], f_vec)

    _issue(0, 0)

    def outer(k2, carry):
        for b in range(NBUF):
            i = k2 * NBUF + b

            @pl.when(i < nchunks)
            def _():
                _wait(b)

                @pl.when(i + 1 < nchunks)
                def _():
                    _issue(1 - b, i + 1)

                _process(b)
        return carry

    lax.fori_loop(0, MAXCH // NBUF, outer, 0)
    pltpu.sync_copy(acc, part_hbm.at[wid])


@functools.partial(
    pl.kernel,
    out_type=jax.ShapeDtypeStruct((ACC,), jnp.float32),
    mesh=_mesh,
    compiler_params=_params,
    scratch_types=[
        pltpu.VMEM((NW * 8 * AW,), jnp.float32),
        pltpu.VMEM((8 * AW,), jnp.float32),
        pltpu.SemaphoreType.DMA,
    ],
)
def _phase2(part_hbm, out_hbm, pbuf, obuf, sem):
    wid = lax.axis_index("s") * 2 + lax.axis_index("c")
    base = wid * (8 * AW)
    descs = []
    for p in range(NW):
        descs.append(pltpu.async_copy(
            part_hbm.at[p, pl.ds(base, 8 * AW)],
            pbuf.at[pl.ds(p * 8 * AW, 8 * AW)], sem))
    for dsx in descs:
        dsx.wait()
    zeros16 = jnp.zeros((16,), jnp.float32)
    for i in range(8 * AW // 16):
        obuf[pl.ds(16 * i, 16)] = zeros16

    def p_body(p, carry):
        pb = p * (8 * AW)
        for i in range(8 * AW // 16):
            plsc.addupdate(obuf.at[pl.ds(16 * i, 16)],
                           pbuf[pl.ds(pb + 16 * i, 16)])
        return carry

    lax.fori_loop(0, NW, p_body, 0)
    pltpu.sync_copy(obuf, out_hbm.at[pl.ds(base, 8 * AW)])


def kernel(h, node_feat_discrete, segment_ids, W, b, scale_table, bias_table,
           mean, std):
    h = h.astype(jnp.float32)
    seg = segment_ids.astype(jnp.int32)
    typ = node_feat_discrete.astype(jnp.int32)
    std0 = std.astype(jnp.float32)[0]
    a_small = std0 * scale_table.astype(jnp.float32)[:, 0]
    c_small = (a_small * b.astype(jnp.float32)[0]
               + std0 * bias_table.astype(jnp.float32)[:, 0]
               + mean.astype(jnp.float32)[0])
    a_pad = jnp.zeros((D,), jnp.float32).at[:U].set(a_small)
    c_pad = jnp.zeros((D,), jnp.float32).at[:U].set(c_small)
    partials = _phase1(h.reshape(-1), seg, typ,
                       W.astype(jnp.float32).reshape(-1), a_pad, c_pad)
    outflat = _phase2(partials)
    r = outflat.reshape(S, AW)
    return (r[:, :D], r[:, D:D + 1])
